# baseline (speedup 1.0000x reference)
import functools

import jax
import jax.numpy as jnp
from jax import lax
from jax.experimental import pallas as pl
from jax.experimental.pallas import tpu as pltpu
from jax.experimental.pallas import tpu_sc as plsc

N = 10000
NP = 10112
RPT = 632
E = 320000
NW = 32
K = 128
NCHUNK = 80
EW = E // NW
PADW = NCHUNK * K - EW
EP = NW * NCHUNK * K
NPOOL = NW * 3 * K
NCHB = 3
NG = 256
NSEGP = 384
SPT = 24
D_IN = 128
HID = 64
EMB = 32
NB = 2528
FW = 128


def _mesh():
    return plsc.VectorSubcoreMesh(core_axis_name="c", subcore_axis_name="s",
                                  num_cores=2, num_subcores=16)


def _zero_acc(zbuf, acc, base, rows):
    full, rem = rows // K, rows % K
    for r in range(full):
        pltpu.sync_copy(zbuf, acc.at[pl.ds(base + r * K, K)])
    if rem:
        pltpu.sync_copy(zbuf.at[pl.ds(0, rem)], acc.at[pl.ds(base + full * K, rem)])


def _make_deg_counts():
    @functools.partial(
        pl.kernel,
        mesh=_mesh(),
        out_type=(
            jax.ShapeDtypeStruct((2, NP, FW), jnp.float32),
            jax.ShapeDtypeStruct((2, NSEGP, FW), jnp.float32),
        ),
        scratch_types=[
            pltpu.VMEM((NCHUNK, K), jnp.int32),
            pltpu.VMEM((NCHB, K), jnp.int32),
            pltpu.VMEM((K, FW), jnp.float32),
            pltpu.VMEM_SHARED((NP, FW), jnp.float32),
            pltpu.VMEM_SHARED((NSEGP, FW), jnp.float32),
        ],
    )
    def deg_kernel(dsts_hbm, bidx_hbm, ones_hbm, zeros_hbm, deg_out, cnt_out,
                   dstv, bv, onesv, acc_d, acc_c):
        c = lax.axis_index("c")
        s = lax.axis_index("s")
        wid = s * 2 + c
        pltpu.sync_copy(dsts_hbm.at[wid], dstv)
        pltpu.sync_copy(bidx_hbm.at[wid], bv)
        pltpu.sync_copy(zeros_hbm, onesv)
        _zero_acc(onesv, acc_d, s * RPT, RPT)
        _zero_acc(onesv, acc_c, s * SPT, SPT)
        pltpu.sync_copy(ones_hbm, onesv)
        plsc.subcore_barrier()

        def body(j, carry):
            pltpu.sync_copy(onesv, acc_d.at[dstv.at[j]], add=True)
            return carry

        lax.fori_loop(0, NCHUNK, body, 0)
        for j in range(NCHB):
            pltpu.sync_copy(onesv, acc_c.at[bv.at[j]], add=True)
        plsc.subcore_barrier()
        pltpu.sync_copy(acc_d.at[pl.ds(s * RPT, RPT)],
                        deg_out.at[c].at[pl.ds(s * RPT, RPT)])
        pltpu.sync_copy(acc_c.at[pl.ds(s * SPT, SPT)],
                        cnt_out.at[c].at[pl.ds(s * SPT, SPT)])

    return deg_kernel


def _make_scatter():
    @functools.partial(
        pl.kernel,
        mesh=_mesh(),
        out_type=jax.ShapeDtypeStruct((2, NP, FW), jnp.float32),
        scratch_types=[
            pltpu.VMEM((NCHUNK, K), jnp.int32),
            pltpu.VMEM((NCHUNK, K), jnp.int32),
            pltpu.VMEM((K, FW), jnp.float32),
            pltpu.VMEM_SHARED((NP, FW), jnp.float32),
        ],
    )
    def scat_kernel(g_hbm, srcs_hbm, dsts_hbm, zeros_hbm, out_hbm,
                    srcv, dstv, rows, acc):
        c = lax.axis_index("c")
        s = lax.axis_index("s")
        wid = s * 2 + c
        pltpu.sync_copy(srcs_hbm.at[wid], srcv)
        pltpu.sync_copy(dsts_hbm.at[wid], dstv)
        pltpu.sync_copy(zeros_hbm, rows)
        _zero_acc(rows, acc, s * RPT, RPT)
        plsc.subcore_barrier()

        def body(j, carry):
            pltpu.sync_copy(g_hbm.at[srcv.at[j]], rows)
            pltpu.sync_copy(rows, acc.at[dstv.at[j]], add=True)
            return carry

        lax.fori_loop(0, NCHUNK, body, 0)
        plsc.subcore_barrier()
        pltpu.sync_copy(acc.at[pl.ds(s * RPT, RPT)],
                        out_hbm.at[c].at[pl.ds(s * RPT, RPT)])

    return scat_kernel


def _make_pool():
    @functools.partial(
        pl.kernel,
        mesh=_mesh(),
        out_type=jax.ShapeDtypeStruct((2, NSEGP, FW), jnp.float32),
        scratch_types=[
            pltpu.VMEM((NCHB, K), jnp.int32),
            pltpu.VMEM((NCHB, K), jnp.int32),
            pltpu.VMEM((K, FW), jnp.float32),
            pltpu.VMEM_SHARED((NSEGP, FW), jnp.float32),
        ],
    )
    def pool_kernel(x_hbm, nid_hbm, bidx_hbm, zeros_hbm, out_hbm,
                    nidv, bv, rows, acc):
        c = lax.axis_index("c")
        s = lax.axis_index("s")
        wid = s * 2 + c
        pltpu.sync_copy(nid_hbm.at[wid], nidv)
        pltpu.sync_copy(bidx_hbm.at[wid], bv)
        pltpu.sync_copy(zeros_hbm, rows)
        _zero_acc(rows, acc, s * SPT, SPT)
        plsc.subcore_barrier()
        for j in range(NCHB):
            pltpu.sync_copy(x_hbm.at[nidv.at[j]], rows)
            pltpu.sync_copy(rows, acc.at[bv.at[j]], add=True)
        plsc.subcore_barrier()
        pltpu.sync_copy(acc.at[pl.ds(s * SPT, SPT)],
                        out_hbm.at[c].at[pl.ds(s * SPT, SPT)])

    return pool_kernel


def _dinv(d0, d1):
    return lax.rsqrt(1.0 + d0[:, :1] + d1[:, :1])


def _t_in_body(x_ref, d0_ref, d1_ref, w_ref, o_ref):
    o_ref[...] = _dinv(d0_ref[...], d1_ref[...]) * jnp.dot(
        x_ref[...], w_ref[...], preferred_element_type=jnp.float32)


def _t_mid_body(a0_ref, a1_ref, g_ref, d0_ref, d1_ref, b_ref, w_ref, o_ref):
    dinv = _dinv(d0_ref[...], d1_ref[...])
    xn = jnp.maximum(dinv * (a0_ref[...] + a1_ref[...] + g_ref[...]) + b_ref[...], 0.0)
    o_ref[...] = dinv * jnp.dot(xn, w_ref[...], preferred_element_type=jnp.float32)


def _t_last_body(a0_ref, a1_ref, g_ref, d0_ref, d1_ref, b_ref, o_ref):
    dinv = _dinv(d0_ref[...], d1_ref[...])
    o_ref[...] = jnp.maximum(
        dinv * (a0_ref[...] + a1_ref[...] + g_ref[...]) + b_ref[...], 0.0)


def _t_head_body(s0_ref, s1_ref, c0_ref, c1_ref, w1_ref, b1_ref, w2_ref, b2_ref,
                 o_ref):
    cnt = jnp.maximum(c0_ref[:, :1] + c1_ref[:, :1], 1.0)
    emb = (s0_ref[:, :EMB] + s1_ref[:, :EMB]) / cnt
    h = jnp.maximum(
        jnp.dot(emb, w1_ref[...], preferred_element_type=jnp.float32) + b1_ref[...],
        0.0)
    o = jnp.dot(h, w2_ref[...], preferred_element_type=jnp.float32) + b2_ref[...]
    o_ref[...] = o[:NG]


def _nb(F):
    return pl.BlockSpec((NB, F), lambda i: (i, 0))


def _const(shape):
    return pl.BlockSpec(shape, lambda i: (0, 0))


def kernel(x, edge_index, batch, W1, b1, W2, b2, W3, b3, Wc1, bc1, Wc2, bc2):
    src = edge_index[0].astype(jnp.int32)
    dst = edge_index[1].astype(jnp.int32)
    batch_i = batch.astype(jnp.int32)

    pad_e = jnp.full((EP - E,), NP - 1, jnp.int32)
    srcs = jnp.concatenate([src, pad_e]).reshape(NW, NCHUNK, K)
    dsts = jnp.concatenate([dst, pad_e]).reshape(NW, NCHUNK, K)
    bidx = jnp.concatenate(
        [batch_i,
         NG + (jnp.arange(NPOOL - N, dtype=jnp.int32) % (NSEGP - NG))]).reshape(
        NW, NCHB, K)
    nid = jnp.concatenate(
        [jnp.arange(N, dtype=jnp.int32),
         jnp.full((NPOOL - N,), NP - 1, jnp.int32)]).reshape(NW, NCHB, K)
    x_p = jnp.zeros((NP, D_IN), jnp.float32).at[:N].set(x)

    zerosF = jnp.zeros((K, FW), jnp.float32)
    onesF = jnp.ones((K, FW), jnp.float32)

    W1p = jnp.zeros((D_IN, FW), jnp.float32).at[:, :HID].set(W1)
    W2p = jnp.zeros((FW, FW), jnp.float32).at[:HID, :HID].set(W2)
    W3p = jnp.zeros((FW, FW), jnp.float32).at[:HID, :EMB].set(W3)
    b1p = jnp.zeros((1, FW), jnp.float32).at[0, :HID].set(b1)
    b2p = jnp.zeros((1, FW), jnp.float32).at[0, :HID].set(b2)
    b3p = jnp.zeros((1, FW), jnp.float32).at[0, :EMB].set(b3)
    bc1r, bc2r = bc1.reshape(1, 16), bc2.reshape(1, 1)

    d_part, c_part = _make_deg_counts()(dsts, bidx, onesF, zerosF)
    d0, d1 = d_part[0, :, :8], d_part[1, :, :8]
    c0, c1 = c_part[0, :, :8], c_part[1, :, :8]

    scat = _make_scatter()

    g1 = pl.pallas_call(
        _t_in_body, grid=(NP // NB,),
        in_specs=[_nb(D_IN), _nb(8), _nb(8), _const((D_IN, FW))],
        out_specs=_nb(FW),
        out_shape=jax.ShapeDtypeStruct((NP, FW), jnp.float32),
    )(x_p, d0, d1, W1p)
    a = scat(g1, srcs, dsts, zerosF)

    g2 = pl.pallas_call(
        _t_mid_body, grid=(NP // NB,),
        in_specs=[_nb(FW), _nb(FW), _nb(FW), _nb(8), _nb(8),
                  _const((1, FW)), _const((FW, FW))],
        out_specs=_nb(FW),
        out_shape=jax.ShapeDtypeStruct((NP, FW), jnp.float32),
    )(a[0], a[1], g1, d0, d1, b1p, W2p)
    a = scat(g2, srcs, dsts, zerosF)

    g3 = pl.pallas_call(
        _t_mid_body, grid=(NP // NB,),
        in_specs=[_nb(FW), _nb(FW), _nb(FW), _nb(8), _nb(8),
                  _const((1, FW)), _const((FW, FW))],
        out_specs=_nb(FW),
        out_shape=jax.ShapeDtypeStruct((NP, FW), jnp.float32),
    )(a[0], a[1], g2, d0, d1, b2p, W3p)
    a = scat(g3, srcs, dsts, zerosF)

    x4 = pl.pallas_call(
        _t_last_body, grid=(NP // NB,),
        in_specs=[_nb(FW), _nb(FW), _nb(FW), _nb(8), _nb(8),
                  _const((1, FW))],
        out_specs=_nb(FW),
        out_shape=jax.ShapeDtypeStruct((NP, FW), jnp.float32),
    )(a[0], a[1], g3, d0, d1, b3p)

    p = _make_pool()(x4, nid, bidx, zerosF)
    out = pl.pallas_call(
        _t_head_body, grid=(1,),
        in_specs=[_const((NSEGP, FW)), _const((NSEGP, FW)),
                  _const((NSEGP, 8)), _const((NSEGP, 8)),
                  _const((EMB, 16)), _const((1, 16)),
                  _const((16, 1)), _const((1, 1))],
        out_specs=_const((NG, 1)),
        out_shape=jax.ShapeDtypeStruct((NG, 1), jnp.float32),
    )(p[0], p[1], c0, c1, Wc1, bc1r, Wc2, bc2r)
    return out


# speedup vs baseline: 1.9578x; 1.9578x over previous
import functools

import jax
import jax.numpy as jnp
from jax import lax
from jax.experimental import pallas as pl
from jax.experimental.pallas import tpu as pltpu
from jax.experimental.pallas import tpu_sc as plsc

N = 10000
NP = 10112
RPT = 632
E = 320000
NW = 32
K = 128
NCHUNK = 79
EW = E // NW
PADW = NCHUNK * K - EW
EP = NW * NCHUNK * K
NPOOL = NW * 3 * K
NCHB = 3
NG = 256
NSEGP = 384
SPT = 24
D_IN = 128
HID = 64
EMB = 32
NB = 2528
FW = 128


def _mesh():
    return plsc.VectorSubcoreMesh(core_axis_name="c", subcore_axis_name="s",
                                  num_cores=2, num_subcores=16)


def _zero_acc(zbuf, acc, base, rows):
    full, rem = rows // K, rows % K
    for r in range(full):
        pltpu.sync_copy(zbuf, acc.at[pl.ds(base + r * K, K)])
    if rem:
        pltpu.sync_copy(zbuf.at[pl.ds(0, rem)], acc.at[pl.ds(base + full * K, rem)])


def _make_deg_counts():
    @functools.partial(
        pl.kernel,
        mesh=_mesh(),
        out_type=(
            jax.ShapeDtypeStruct((2, NP, FW), jnp.float32),
            jax.ShapeDtypeStruct((2, NSEGP, FW), jnp.float32),
        ),
        scratch_types=[
            pltpu.VMEM((NCHUNK, K), jnp.int32),
            pltpu.VMEM((NCHB, K), jnp.int32),
            pltpu.VMEM((K, FW), jnp.float32),
            pltpu.VMEM_SHARED((NP, FW), jnp.float32),
            pltpu.VMEM_SHARED((NSEGP, FW), jnp.float32),
        ],
    )
    def deg_kernel(dsts_hbm, bidx_hbm, ones_hbm, zeros_hbm, deg_out, cnt_out,
                   dstv, bv, onesv, acc_d, acc_c):
        c = lax.axis_index("c")
        s = lax.axis_index("s")
        wid = s * 2 + c
        pltpu.sync_copy(dsts_hbm.at[wid], dstv)
        pltpu.sync_copy(bidx_hbm.at[wid], bv)
        pltpu.sync_copy(zeros_hbm, onesv)
        _zero_acc(onesv, acc_d, s * RPT, RPT)
        _zero_acc(onesv, acc_c, s * SPT, SPT)
        pltpu.sync_copy(ones_hbm, onesv)
        plsc.subcore_barrier()

        def body(j, carry):
            pltpu.sync_copy(onesv, acc_d.at[dstv.at[j]], add=True)
            return carry

        lax.fori_loop(0, NCHUNK, body, 0)
        for j in range(NCHB):
            pltpu.sync_copy(onesv, acc_c.at[bv.at[j]], add=True)
        plsc.subcore_barrier()
        pltpu.sync_copy(acc_d.at[pl.ds(s * RPT, RPT)],
                        deg_out.at[c].at[pl.ds(s * RPT, RPT)])
        pltpu.sync_copy(acc_c.at[pl.ds(s * SPT, SPT)],
                        cnt_out.at[c].at[pl.ds(s * SPT, SPT)])

    return deg_kernel


def _make_scatter():
    H0 = (NCHUNK + 1) // 2
    H1 = NCHUNK - H0

    @functools.partial(
        pl.kernel,
        mesh=_mesh(),
        out_type=jax.ShapeDtypeStruct((2, NP, FW), jnp.float32),
        scratch_types=[
            pltpu.VMEM((H0, K), jnp.int32),
            pltpu.VMEM((H0, K), jnp.int32),
            pltpu.VMEM((K, FW), jnp.float32),
            pltpu.VMEM((K, FW), jnp.float32),
            pltpu.VMEM_SHARED((NP, FW), jnp.float32),
            pltpu.SemaphoreType.DMA,
            pltpu.SemaphoreType.DMA,
        ],
    )
    def scat_kernel(g_hbm, srcs_hbm, dsts_hbm, zeros_hbm, out_hbm,
                    srcv, dstv, rows0, rows1, acc, sem0, sem1):
        c = lax.axis_index("c")
        s = lax.axis_index("s")
        wid = s * 2 + c
        pltpu.sync_copy(zeros_hbm, rows0)
        _zero_acc(rows0, acc, s * RPT, RPT)
        plsc.subcore_barrier()

        def make_body(nh):
            def body(i, carry):
                j = 2 * i

                @pl.when(j + 1 < nh)
                def _():
                    pltpu.async_copy(g_hbm.at[srcv.at[j + 1]], rows1, sem1)

                pltpu.make_async_copy(g_hbm.at[srcv.at[j]], rows0, sem0).wait()
                pltpu.sync_copy(rows0, acc.at[dstv.at[j]], add=True)

                @pl.when(j + 2 < nh)
                def _():
                    pltpu.async_copy(g_hbm.at[srcv.at[j + 2]], rows0, sem0)

                @pl.when(j + 1 < nh)
                def _():
                    pltpu.make_async_copy(
                        g_hbm.at[srcv.at[j + 1]], rows1, sem1).wait()
                    pltpu.sync_copy(rows1, acc.at[dstv.at[j + 1]], add=True)

                return carry
            return body

        for h, nh in ((0, H0), (1, H1)):
            pltpu.sync_copy(srcs_hbm.at[wid].at[pl.ds(h * H0, nh)], srcv.at[pl.ds(0, nh)])
            pltpu.sync_copy(dsts_hbm.at[wid].at[pl.ds(h * H0, nh)], dstv.at[pl.ds(0, nh)])
            pltpu.async_copy(g_hbm.at[srcv.at[0]], rows0, sem0)
            lax.fori_loop(0, (nh + 1) // 2, make_body(nh), 0)

        plsc.subcore_barrier()
        pltpu.sync_copy(acc.at[pl.ds(s * RPT, RPT)],
                        out_hbm.at[c].at[pl.ds(s * RPT, RPT)])

    return scat_kernel


def _make_pool():
    @functools.partial(
        pl.kernel,
        mesh=_mesh(),
        out_type=jax.ShapeDtypeStruct((2, NSEGP, FW), jnp.float32),
        scratch_types=[
            pltpu.VMEM((NCHB, K), jnp.int32),
            pltpu.VMEM((NCHB, K), jnp.int32),
            pltpu.VMEM((K, FW), jnp.float32),
            pltpu.VMEM_SHARED((NSEGP, FW), jnp.float32),
        ],
    )
    def pool_kernel(x_hbm, nid_hbm, bidx_hbm, zeros_hbm, out_hbm,
                    nidv, bv, rows, acc):
        c = lax.axis_index("c")
        s = lax.axis_index("s")
        wid = s * 2 + c
        pltpu.sync_copy(nid_hbm.at[wid], nidv)
        pltpu.sync_copy(bidx_hbm.at[wid], bv)
        pltpu.sync_copy(zeros_hbm, rows)
        _zero_acc(rows, acc, s * SPT, SPT)
        plsc.subcore_barrier()
        for j in range(NCHB):
            pltpu.sync_copy(x_hbm.at[nidv.at[j]], rows)
            pltpu.sync_copy(rows, acc.at[bv.at[j]], add=True)
        plsc.subcore_barrier()
        pltpu.sync_copy(acc.at[pl.ds(s * SPT, SPT)],
                        out_hbm.at[c].at[pl.ds(s * SPT, SPT)])

    return pool_kernel


def _dinv(d0, d1):
    return lax.rsqrt(1.0 + d0[:, :1] + d1[:, :1])


def _t_in_body(x_ref, d0_ref, d1_ref, w_ref, o_ref):
    o_ref[...] = _dinv(d0_ref[...], d1_ref[...]) * jnp.dot(
        x_ref[...], w_ref[...], preferred_element_type=jnp.float32)


def _t_mid_body(a0_ref, a1_ref, g_ref, d0_ref, d1_ref, b_ref, w_ref, o_ref):
    dinv = _dinv(d0_ref[...], d1_ref[...])
    xn = jnp.maximum(dinv * (a0_ref[...] + a1_ref[...] + g_ref[...]) + b_ref[...], 0.0)
    o_ref[...] = dinv * jnp.dot(xn, w_ref[...], preferred_element_type=jnp.float32)


def _t_last_body(a0_ref, a1_ref, g_ref, d0_ref, d1_ref, b_ref, o_ref):
    dinv = _dinv(d0_ref[...], d1_ref[...])
    o_ref[...] = jnp.maximum(
        dinv * (a0_ref[...] + a1_ref[...] + g_ref[...]) + b_ref[...], 0.0)


def _t_head_body(s0_ref, s1_ref, c0_ref, c1_ref, w1_ref, b1_ref, w2_ref, b2_ref,
                 o_ref):
    cnt = jnp.maximum(c0_ref[:, :1] + c1_ref[:, :1], 1.0)
    emb = (s0_ref[:, :EMB] + s1_ref[:, :EMB]) / cnt
    h = jnp.maximum(
        jnp.dot(emb, w1_ref[...], preferred_element_type=jnp.float32) + b1_ref[...],
        0.0)
    o = jnp.dot(h, w2_ref[...], preferred_element_type=jnp.float32) + b2_ref[...]
    o_ref[...] = o[:NG]


def _nb(F):
    return pl.BlockSpec((NB, F), lambda i: (i, 0))


def _const(shape):
    return pl.BlockSpec(shape, lambda i: (0, 0))


def kernel(x, edge_index, batch, W1, b1, W2, b2, W3, b3, Wc1, bc1, Wc2, bc2):
    src = edge_index[0].astype(jnp.int32)
    dst = edge_index[1].astype(jnp.int32)
    batch_i = batch.astype(jnp.int32)

    pad_e = jnp.full((EP - E,), NP - 1, jnp.int32)
    srcs = jnp.concatenate([src, pad_e]).reshape(NW, NCHUNK, K)
    dsts = jnp.concatenate([dst, pad_e]).reshape(NW, NCHUNK, K)
    bidx = jnp.concatenate(
        [batch_i,
         NG + (jnp.arange(NPOOL - N, dtype=jnp.int32) % (NSEGP - NG))]).reshape(
        NW, NCHB, K)
    nid = jnp.concatenate(
        [jnp.arange(N, dtype=jnp.int32),
         jnp.full((NPOOL - N,), NP - 1, jnp.int32)]).reshape(NW, NCHB, K)
    x_p = jnp.zeros((NP, D_IN), jnp.float32).at[:N].set(x)

    zerosF = jnp.zeros((K, FW), jnp.float32)
    onesF = jnp.ones((K, FW), jnp.float32)

    W1p = jnp.zeros((D_IN, FW), jnp.float32).at[:, :HID].set(W1)
    W2p = jnp.zeros((FW, FW), jnp.float32).at[:HID, :HID].set(W2)
    W3p = jnp.zeros((FW, FW), jnp.float32).at[:HID, :EMB].set(W3)
    b1p = jnp.zeros((1, FW), jnp.float32).at[0, :HID].set(b1)
    b2p = jnp.zeros((1, FW), jnp.float32).at[0, :HID].set(b2)
    b3p = jnp.zeros((1, FW), jnp.float32).at[0, :EMB].set(b3)
    bc1r, bc2r = bc1.reshape(1, 16), bc2.reshape(1, 1)

    d_part, c_part = _make_deg_counts()(dsts, bidx, onesF, zerosF)
    d0, d1 = d_part[0, :, :8], d_part[1, :, :8]
    c0, c1 = c_part[0, :, :8], c_part[1, :, :8]

    scat = _make_scatter()

    g1 = pl.pallas_call(
        _t_in_body, grid=(NP // NB,),
        in_specs=[_nb(D_IN), _nb(8), _nb(8), _const((D_IN, FW))],
        out_specs=_nb(FW),
        out_shape=jax.ShapeDtypeStruct((NP, FW), jnp.float32),
    )(x_p, d0, d1, W1p)
    a = scat(g1, srcs, dsts, zerosF)

    g2 = pl.pallas_call(
        _t_mid_body, grid=(NP // NB,),
        in_specs=[_nb(FW), _nb(FW), _nb(FW), _nb(8), _nb(8),
                  _const((1, FW)), _const((FW, FW))],
        out_specs=_nb(FW),
        out_shape=jax.ShapeDtypeStruct((NP, FW), jnp.float32),
    )(a[0], a[1], g1, d0, d1, b1p, W2p)
    a = scat(g2, srcs, dsts, zerosF)

    g3 = pl.pallas_call(
        _t_mid_body, grid=(NP // NB,),
        in_specs=[_nb(FW), _nb(FW), _nb(FW), _nb(8), _nb(8),
                  _const((1, FW)), _const((FW, FW))],
        out_specs=_nb(FW),
        out_shape=jax.ShapeDtypeStruct((NP, FW), jnp.float32),
    )(a[0], a[1], g2, d0, d1, b2p, W3p)
    a = scat(g3, srcs, dsts, zerosF)

    x4 = pl.pallas_call(
        _t_last_body, grid=(NP // NB,),
        in_specs=[_nb(FW), _nb(FW), _nb(FW), _nb(8), _nb(8),
                  _const((1, FW))],
        out_specs=_nb(FW),
        out_shape=jax.ShapeDtypeStruct((NP, FW), jnp.float32),
    )(a[0], a[1], g3, d0, d1, b3p)

    p = _make_pool()(x4, nid, bidx, zerosF)
    out = pl.pallas_call(
        _t_head_body, grid=(1,),
        in_specs=[_const((NSEGP, FW)), _const((NSEGP, FW)),
                  _const((NSEGP, 8)), _const((NSEGP, 8)),
                  _const((EMB, 16)), _const((1, 16)),
                  _const((16, 1)), _const((1, 1))],
        out_specs=_const((NG, 1)),
        out_shape=jax.ShapeDtypeStruct((NG, 1), jnp.float32),
    )(p[0], p[1], c0, c1, Wc1, bc1r, Wc2, bc2r)
    return out
